# Initial kernel scaffold; baseline (speedup 1.0000x reference)
import jax
import jax.numpy as jnp
from jax.experimental import pallas as pl


def kernel(x, y, alpha, beta, sigma, alpha_mask, AllSPL):
    raise NotImplementedError("write your pallas kernel here")


# speedup vs baseline: 12.4700x; 12.4700x over previous
import functools

import jax
import jax.numpy as jnp
import numpy as np
from jax import lax
from jax.experimental import pallas as pl
from jax.experimental.pallas import tpu as pltpu
from jax.experimental.pallas import tpu_sc as plsc

N = 2000000
NW = 32
L = 16
VPB = 126
BLK = VPB * L
NBLK = 31
MAIN_ROWS = NW * NBLK * BLK
TAIL_VECS = (N - MAIN_ROWS) // L

_SC0 = np.float32(111.32 * 0.772)
_SC1 = np.float32(110.574)


def _make_kernel():
    mesh = plsc.VectorSubcoreMesh(core_axis_name="c", subcore_axis_name="s")

    @functools.partial(
        pl.kernel,
        out_type=jax.ShapeDtypeStruct((N,), jnp.float32),
        mesh=mesh,
        compiler_params=pltpu.CompilerParams(needs_layout_passes=False),
        scratch_types=[
            pltpu.VMEM((BLK * 7,), jnp.float32),
            pltpu.VMEM((BLK * 7,), jnp.float32),
            pltpu.VMEM((BLK,), jnp.float32),
            pltpu.VMEM((64,), jnp.float32),
            pltpu.VMEM((64,), jnp.float32),
            pltpu.VMEM((64,), jnp.float32),
            pltpu.VMEM((32,), jnp.float32),
        ],
    )
    def sc_kernel(xf, yf, af, mf, sf, pf, out, xbuf, ybuf, obuf, atbl, mtbl,
                  stbl, pbuf):
        wid = lax.axis_index("s") * 2 + lax.axis_index("c")

        pltpu.sync_copy(af, atbl)
        pltpu.sync_copy(mf, mtbl)
        pltpu.sync_copy(sf, stbl)
        pltpu.sync_copy(pf, pbuf)

        for t in range(4):
            sl = pl.ds(t * L, L)
            atbl[sl] = jnp.where(mtbl[sl] != 0.0, atbl[sl], 0.0)

        beta = pbuf[pl.ds(0, L)]
        sigma = pbuf[pl.ds(L, L)]
        inv2s2 = 1.0 / (2.0 * sigma * sigma)
        cnorm = beta * inv2s2 * np.float32(1.0 / np.pi)

        i7 = lax.iota(jnp.int32, L) * 7

        def compute_vec(base):
            idx = i7 + base
            x0 = plsc.load_gather(xbuf, [idx])
            x1 = plsc.load_gather(xbuf, [idx + 1])
            x2 = plsc.load_gather(xbuf, [idx + 2])
            x3 = plsc.load_gather(xbuf, [idx + 3])
            x4 = plsc.load_gather(xbuf, [idx + 4])
            x5 = plsc.load_gather(xbuf, [idx + 5])
            x6 = plsc.load_gather(xbuf, [idx + 6])
            y0 = plsc.load_gather(ybuf, [idx])
            y1 = plsc.load_gather(ybuf, [idx + 1])
            y2 = plsc.load_gather(ybuf, [idx + 2])
            y3 = plsc.load_gather(ybuf, [idx + 3])
            y4 = plsc.load_gather(ybuf, [idx + 4])
            y5 = plsc.load_gather(ybuf, [idx + 5])
            y6 = plsc.load_gather(ybuf, [idx + 6])

            aidx = x1.astype(jnp.int32) * 8 + y1.astype(jnp.int32)
            alphas = plsc.load_gather(atbl, [aidx])
            sidx = x4.astype(jnp.int32) * 8 + y4.astype(jnp.int32)
            spl = plsc.load_gather(stbl, [sidx])

            tds = jnp.where(x0 > 0.0, x0 - y0, jnp.float32(1.0))
            dlon = (x2 - y2) * _SC0
            dlat = (x3 - y3) * _SC1
            sq = jnp.maximum(dlon * dlon + dlat * dlat, np.float32(1e-12))
            a3 = (x5 + y5 + spl) * np.float32(1e-3)
            nw2 = jnp.where(x6 == y6, sq, a3 * a3)
            itds = 1.0 / tds
            e = jnp.exp(-(beta * tds) - nw2 * inv2s2 * itds)
            return alphas * cnorm * e * itds

        def blk_body(b, carry):
            row0 = pl.multiple_of(wid * (NBLK * BLK) + b * BLK, BLK)
            pltpu.sync_copy(xf.at[pl.ds(row0 * 7, BLK * 7)], xbuf)
            pltpu.sync_copy(yf.at[pl.ds(row0 * 7, BLK * 7)], ybuf)

            def vec_body(v, c):
                obuf[pl.ds(v * L, L)] = compute_vec(v * (7 * L))
                return c

            lax.fori_loop(0, VPB, vec_body, 0)
            pltpu.sync_copy(obuf, out.at[pl.ds(row0, BLK)])
            return carry

        lax.fori_loop(0, NBLK, blk_body, 0)

        @pl.when(wid < TAIL_VECS)
        def _():
            row0 = pl.multiple_of(MAIN_ROWS + wid * L, L)
            pltpu.sync_copy(xf.at[pl.ds(row0 * 7, 7 * L)], xbuf.at[pl.ds(0, 7 * L)])
            pltpu.sync_copy(yf.at[pl.ds(row0 * 7, 7 * L)], ybuf.at[pl.ds(0, 7 * L)])
            obuf[pl.ds(0, L)] = compute_vec(0)
            pltpu.sync_copy(obuf.at[pl.ds(0, L)], out.at[pl.ds(row0, L)])

    return sc_kernel


_KERNEL = _make_kernel()


def kernel(x, y, alpha, beta, sigma, alpha_mask, AllSPL):
    params = jnp.concatenate([
        jnp.full((L,), beta, dtype=jnp.float32),
        jnp.full((L,), sigma, dtype=jnp.float32),
    ])
    return _KERNEL(
        x.reshape(-1),
        y.reshape(-1),
        alpha.reshape(-1),
        alpha_mask.reshape(-1),
        AllSPL.reshape(-1),
        params,
    )


# speedup vs baseline: 12.9919x; 1.0419x over previous
import functools

import jax
import jax.numpy as jnp
import numpy as np
from jax import lax
from jax.experimental import pallas as pl
from jax.experimental.pallas import tpu as pltpu
from jax.experimental.pallas import tpu_sc as plsc

N = 2000000
NW = 32
L = 16
VPB = 126
BLK = VPB * L
NBLK = 31
MAIN_ROWS = NW * NBLK * BLK
TAIL_VECS = (N - MAIN_ROWS) // L

_SC0 = np.float32(111.32 * 0.772)
_SC1 = np.float32(110.574)


def _make_kernel():
    mesh = plsc.VectorSubcoreMesh(core_axis_name="c", subcore_axis_name="s")

    @functools.partial(
        pl.kernel,
        out_type=jax.ShapeDtypeStruct((N,), jnp.float32),
        mesh=mesh,
        compiler_params=pltpu.CompilerParams(needs_layout_passes=False),
        scratch_types=[
            pltpu.VMEM((BLK * 7,), jnp.float32),
            pltpu.VMEM((BLK * 7,), jnp.float32),
            pltpu.VMEM((BLK,), jnp.float32),
            pltpu.VMEM((64,), jnp.float32),
            pltpu.VMEM((64,), jnp.float32),
            pltpu.VMEM((64,), jnp.float32),
            pltpu.VMEM((32,), jnp.float32),
        ],
    )
    def sc_kernel(xf, yf, af, mf, sf, pf, out, xbuf, ybuf, obuf, atbl, mtbl,
                  stbl, pbuf):
        wid = lax.axis_index("s") * 2 + lax.axis_index("c")

        pltpu.sync_copy(af, atbl)
        pltpu.sync_copy(mf, mtbl)
        pltpu.sync_copy(sf, stbl)
        pltpu.sync_copy(pf, pbuf)

        for t in range(4):
            sl = pl.ds(t * L, L)
            atbl[sl] = jnp.where(mtbl[sl] != 0.0, atbl[sl], 0.0)

        beta = pbuf[pl.ds(0, L)]
        sigma = pbuf[pl.ds(L, L)]
        inv2s2 = 1.0 / (2.0 * sigma * sigma)
        cnorm = beta * inv2s2 * np.float32(1.0 / np.pi)

        i7 = lax.iota(jnp.int32, L) * 7

        def compute_vec(base):
            idx = i7 + base
            x0 = plsc.load_gather(xbuf, [idx])
            x1 = plsc.load_gather(xbuf, [idx + 1])
            x2 = plsc.load_gather(xbuf, [idx + 2])
            x3 = plsc.load_gather(xbuf, [idx + 3])
            x4 = plsc.load_gather(xbuf, [idx + 4])
            x5 = plsc.load_gather(xbuf, [idx + 5])
            x6 = plsc.load_gather(xbuf, [idx + 6])
            y0 = plsc.load_gather(ybuf, [idx])
            y1 = plsc.load_gather(ybuf, [idx + 1])
            y2 = plsc.load_gather(ybuf, [idx + 2])
            y3 = plsc.load_gather(ybuf, [idx + 3])
            y4 = plsc.load_gather(ybuf, [idx + 4])
            y5 = plsc.load_gather(ybuf, [idx + 5])
            y6 = plsc.load_gather(ybuf, [idx + 6])

            aidx = x1.astype(jnp.int32) * 8 + y1.astype(jnp.int32)
            alphas = plsc.load_gather(atbl, [aidx])
            sidx = x4.astype(jnp.int32) * 8 + y4.astype(jnp.int32)
            spl = plsc.load_gather(stbl, [sidx])

            tds = jnp.where(x0 > 0.0, x0 - y0, jnp.float32(1.0))
            dlon = (x2 - y2) * _SC0
            dlat = (x3 - y3) * _SC1
            sq = jnp.maximum(dlon * dlon + dlat * dlat, np.float32(1e-12))
            a3 = (x5 + y5 + spl) * np.float32(1e-3)
            nw2 = jnp.where(x6 == y6, sq, a3 * a3)
            itds = 1.0 / tds
            e = jnp.exp(-(beta * tds) - nw2 * inv2s2 * itds)
            return alphas * cnorm * e * itds

        def blk_body(b, carry):
            row0 = pl.multiple_of(wid * (NBLK * BLK) + b * BLK, BLK)
            pltpu.sync_copy(xf.at[pl.ds(row0 * 7, BLK * 7)], xbuf)
            pltpu.sync_copy(yf.at[pl.ds(row0 * 7, BLK * 7)], ybuf)

            @plsc.parallel_loop(0, VPB, step=1, unroll=4)
            def vec_body(v):
                obuf[pl.ds(v * L, L)] = compute_vec(v * (7 * L))
            pltpu.sync_copy(obuf, out.at[pl.ds(row0, BLK)])
            return carry

        lax.fori_loop(0, NBLK, blk_body, 0)

        @pl.when(wid < TAIL_VECS)
        def _():
            row0 = pl.multiple_of(MAIN_ROWS + wid * L, L)
            pltpu.sync_copy(xf.at[pl.ds(row0 * 7, 7 * L)], xbuf.at[pl.ds(0, 7 * L)])
            pltpu.sync_copy(yf.at[pl.ds(row0 * 7, 7 * L)], ybuf.at[pl.ds(0, 7 * L)])
            obuf[pl.ds(0, L)] = compute_vec(0)
            pltpu.sync_copy(obuf.at[pl.ds(0, L)], out.at[pl.ds(row0, L)])

    return sc_kernel


_KERNEL = _make_kernel()


def kernel(x, y, alpha, beta, sigma, alpha_mask, AllSPL):
    params = jnp.concatenate([
        jnp.full((L,), beta, dtype=jnp.float32),
        jnp.full((L,), sigma, dtype=jnp.float32),
    ])
    return _KERNEL(
        x.reshape(-1),
        y.reshape(-1),
        alpha.reshape(-1),
        alpha_mask.reshape(-1),
        AllSPL.reshape(-1),
        params,
    )
